# Initial kernel scaffold; baseline (speedup 1.0000x reference)
import jax
import jax.numpy as jnp
from jax.experimental import pallas as pl


def kernel(entity_embds, rel_embds, pos_h, pos_r, pos_t):
    raise NotImplementedError("write your pallas kernel here")


# speedup vs baseline: 3.1572x; 3.1572x over previous
import functools

import jax
import jax.numpy as jnp
from jax import lax
from jax.experimental import pallas as pl
from jax.experimental.pallas import tpu as pltpu
from jax.experimental.pallas import tpu_sc as plsc

_NUM_ENTITIES = 1000000
_EMB_DIM = 64
_BATCH = 16384
_L = 16
_NC, _NS = 2, 16
_NW = _NC * _NS
_BPW = _BATCH // _NW
_CHUNK = 128
_NCHUNK = _BPW // _CHUNK
_GROUPS = _BPW // _L


def _rsqrt_nr(s):
    i = lax.bitcast_convert_type(s, jnp.int32)
    i = jnp.int32(0x5F3759DF) - lax.shift_right_logical(i, 1)
    y = lax.bitcast_convert_type(i, jnp.float32)
    for _ in range(3):
        y = y * (jnp.float32(1.5) - jnp.float32(0.5) * s * y * y)
    return y


def _normalize_rows(rows, idx2d):
    def body(g, carry):
        j = lax.shift_right_logical(g, 3)
        start = (g & 7) * _L
        idx_vec = idx2d[j, pl.ds(start, _L)]
        base_row = g * _L
        for k in range(_L):
            i = base_row + k
            v0 = rows[i, pl.ds(0 * _L, _L)]
            v1 = rows[i, pl.ds(1 * _L, _L)]
            v2 = rows[i, pl.ds(2 * _L, _L)]
            v3 = rows[i, pl.ds(3 * _L, _L)]
            ss = v0 * v0 + v1 * v1 + v2 * v2 + v3 * v3
            s = jnp.sum(ss)
            y = _rsqrt_nr(s)
            y = jnp.where(idx_vec[k] == _NUM_ENTITIES - 1,
                          jnp.float32(1.0), y)
            rows[i, pl.ds(0 * _L, _L)] = v0 * y
            rows[i, pl.ds(1 * _L, _L)] = v1 * y
            rows[i, pl.ds(2 * _L, _L)] = v2 * y
            rows[i, pl.ds(3 * _L, _L)] = v3 * y
        return carry

    lax.fori_loop(0, _GROUPS, body, 0, unroll=False)


def _make_sc_call():
    mesh = plsc.VectorSubcoreMesh(core_axis_name="c", subcore_axis_name="s")
    out = jax.ShapeDtypeStruct((_BATCH, _EMB_DIM), jnp.float32)

    @functools.partial(
        pl.kernel,
        mesh=mesh,
        out_type=[out, out, out],
        compiler_params=pltpu.CompilerParams(
            needs_layout_passes=False, use_tc_tiling_on_sc=False),
        scratch_types=[
            pltpu.VMEM((_NCHUNK, _CHUNK), jnp.int32),
            pltpu.VMEM((_NCHUNK, _CHUNK), jnp.int32),
            pltpu.VMEM((_NCHUNK, _CHUNK), jnp.int32),
            pltpu.VMEM((_BPW, _EMB_DIM), jnp.float32),
            pltpu.VMEM((_BPW, _EMB_DIM), jnp.float32),
            pltpu.VMEM((_BPW, _EMB_DIM), jnp.float32),
            pltpu.SemaphoreType.DMA,
            pltpu.SemaphoreType.DMA,
            pltpu.SemaphoreType.DMA,
        ],
    )
    def call(ent_hbm, rel_hbm, ph_hbm, pr_hbm, pt_hbm,
             h_out, r_out, t_out,
             idx_h, idx_r, idx_t, rows_h, rows_r, rows_t,
             sem_h, sem_r, sem_t):
        wid = lax.axis_index("s") * _NC + lax.axis_index("c")
        base = wid * _BPW
        for j in range(_NCHUNK):
            off = pl.ds(base + j * _CHUNK, _CHUNK)
            pltpu.sync_copy(ph_hbm.at[off], idx_h.at[j])
            pltpu.sync_copy(pt_hbm.at[off], idx_t.at[j])
            pltpu.sync_copy(pr_hbm.at[off], idx_r.at[j])
        waits = []
        for tbl, idx, rows, sem in ((ent_hbm, idx_h, rows_h, sem_h),
                                    (ent_hbm, idx_t, rows_t, sem_t),
                                    (rel_hbm, idx_r, rows_r, sem_r)):
            ws = []
            for j in range(_NCHUNK):
                dst = rows.at[pl.ds(j * _CHUNK, _CHUNK), :]
                ws.append(pltpu.async_copy(tbl.at[idx.at[j]], dst, sem))
            waits.append(ws)
        for w in waits[0]:
            w.wait()
        _normalize_rows(rows_h, idx_h)
        pltpu.sync_copy(rows_h, h_out.at[pl.ds(base, _BPW), :])
        for w in waits[1]:
            w.wait()
        _normalize_rows(rows_t, idx_t)
        pltpu.sync_copy(rows_t, t_out.at[pl.ds(base, _BPW), :])
        for w in waits[2]:
            w.wait()
        pltpu.sync_copy(rows_r, r_out.at[pl.ds(base, _BPW), :])

    return call


_sc_call = _make_sc_call()


def kernel(entity_embds, rel_embds, pos_h, pos_r, pos_t):
    h, r, t = _sc_call(entity_embds, rel_embds,
                       pos_h.astype(jnp.int32),
                       pos_r.astype(jnp.int32),
                       pos_t.astype(jnp.int32))
    return (h, r, t)
